# baseline (speedup 1.0000x reference)
import functools

import jax
import jax.numpy as jnp
from jax import lax
from jax.experimental import pallas as pl
from jax.experimental.pallas import tpu as pltpu
from jax.experimental.pallas import tpu_sc as plsc

B, L, D = 1024, 200, 128
R = B * L
NW = 32
RPW = R // NW
C = 256
NCHUNK = RPW // C
LANES = 16

_mesh = plsc.VectorSubcoreMesh(core_axis_name="c", subcore_axis_name="s")


@functools.partial(
    pl.kernel,
    mesh=_mesh,
    out_type=jax.ShapeDtypeStruct((R * D,), jnp.float32),
    scratch_types=[
        pltpu.VMEM((C * D,), jnp.float32),
        pltpu.VMEM((C,), jnp.int32),
    ],
    compiler_params=pltpu.CompilerParams(needs_layout_passes=False),
)
def _masked_copy(x_hbm, seq_hbm, out_hbm, buf, seq_v):
    wid = lax.axis_index("s") * 2 + lax.axis_index("c")
    base = wid * RPW
    zeros = jnp.zeros((LANES,), jnp.float32)
    lane = lax.iota(jnp.int32, LANES)

    def chunk_body(ci, carry):
        rbase = base + ci * C
        pltpu.sync_copy(seq_hbm.at[pl.ds(rbase, C)], seq_v)
        pltpu.sync_copy(x_hbm.at[pl.ds(rbase * D, C * D)], buf)

        def grp_body(g, c2):
            svec = seq_v[pl.ds(g * LANES, LANES)]
            mask = svec == 0
            row_part = (g * LANES + lane) * D
            for col in range(D):
                plsc.store_scatter(buf, [row_part + col], zeros, mask=mask)
            return c2

        lax.fori_loop(0, C // LANES, grp_body, 0)
        pltpu.sync_copy(buf, out_hbm.at[pl.ds(rbase * D, C * D)])
        return carry

    lax.fori_loop(0, NCHUNK, chunk_body, 0)


def kernel(x, item_seq):
    xf = x.reshape(R * D)
    seq = item_seq.reshape(R).astype(jnp.int32)
    out = _masked_copy(xf, seq)
    return out.reshape(B, L, D)
